# Initial kernel scaffold; baseline (speedup 1.0000x reference)
import jax
import jax.numpy as jnp
from jax.experimental import pallas as pl


def kernel(sessions, pos_emb):
    raise NotImplementedError("write your pallas kernel here")


# speedup vs baseline: 3.7047x; 3.7047x over previous
import jax
import jax.numpy as jnp
from jax import lax
from jax.experimental import pallas as pl
from jax.experimental.pallas import tpu as pltpu

_BB = 64


def _body(s_ref, p_ref, o_ref):
    T = p_ref.shape[0]
    row = lax.broadcasted_iota(jnp.int32, (T, T), 0)
    col = lax.broadcasted_iota(jnp.int32, (T, T), 1)
    perm = (row + col == T - 1).astype(jnp.float32)
    pos_rev = jnp.dot(perm, p_ref[...], preferred_element_type=jnp.float32)
    o_ref[...] = s_ref[...] + pos_rev[None, :, :]


def kernel(sessions, pos_emb):
    B, T, F = sessions.shape
    grid = (B // _BB,)
    return pl.pallas_call(
        _body,
        grid=grid,
        in_specs=[
            pl.BlockSpec((_BB, T, F), lambda i: (i, 0, 0)),
            pl.BlockSpec((T, F), lambda i: (0, 0)),
        ],
        out_specs=pl.BlockSpec((_BB, T, F), lambda i: (i, 0, 0)),
        out_shape=jax.ShapeDtypeStruct((B, T, F), sessions.dtype),
        compiler_params=pltpu.CompilerParams(
            dimension_semantics=("arbitrary",),
        ),
    )(sessions, pos_emb)


# speedup vs baseline: 22.6201x; 6.1057x over previous
import jax
import jax.numpy as jnp
from jax.experimental import pallas as pl
from jax.experimental.pallas import tpu as pltpu

_TB = 8


def _body(s_ref, p_ref, o_ref):
    jt = pl.program_id(0)
    base = pl.num_programs(0) * _TB - 1 - jt * _TB
    for k in range(_TB):
        prow = p_ref[base - k]
        o_ref[k] = s_ref[k] + jnp.broadcast_to(prow, s_ref.shape[1:])


def kernel(sessions, pos_emb):
    B, T, F = sessions.shape
    st = jnp.transpose(sessions, (1, 2, 0))
    pos3 = pos_emb[:, :, None]
    out_t = pl.pallas_call(
        _body,
        grid=(T // _TB,),
        in_specs=[
            pl.BlockSpec((_TB, F, B), lambda jt: (jt, 0, 0)),
            pl.BlockSpec((T, F, 1), lambda jt: (0, 0, 0)),
        ],
        out_specs=pl.BlockSpec((_TB, F, B), lambda jt: (jt, 0, 0)),
        out_shape=jax.ShapeDtypeStruct((T, F, B), sessions.dtype),
        compiler_params=pltpu.CompilerParams(
            dimension_semantics=("arbitrary",),
        ),
    )(st, pos3)
    return jnp.transpose(out_t, (2, 0, 1))
